# Initial kernel scaffold; baseline (speedup 1.0000x reference)
import jax
import jax.numpy as jnp
from jax.experimental import pallas as pl


def kernel(feat_src, feat_dst, params, edge_index):
    raise NotImplementedError("write your pallas kernel here")


# speedup vs baseline: 47.1227x; 47.1227x over previous
import functools

import jax
import jax.numpy as jnp
from jax import lax
from jax.experimental import pallas as pl
from jax.experimental.pallas import tpu as pltpu
from jax.experimental.pallas import tpu_sc as plsc

D = 128
NH = 8
HD = 16
N_SRC = 10000
N_DST = 10000
E = 320000
FF = 2048

NW = 32
CHUNK = 128
EPW = 10240
E_PAD = NW * EPW
N_ACC = 10112
ZR = N_ACC // 16


def _pre_body(x_ref, w_ref, al_ref, ar_ref, h_ref, el_ref, er_ref):
    h = jnp.dot(x_ref[...], w_ref[...], preferred_element_type=jnp.float32)
    h_ref[...] = h
    el_ref[...] = jnp.dot(h, al_ref[...], preferred_element_type=jnp.float32)
    er_ref[...] = jnp.dot(h, ar_ref[...], preferred_element_type=jnp.float32)


def _gat_pre(x, w_gat, almat, armat):
    nb = 10
    rb = (2 * N_SRC) // nb
    return pl.pallas_call(
        _pre_body,
        grid=(nb,),
        in_specs=[
            pl.BlockSpec((rb, D), lambda i: (i, 0)),
            pl.BlockSpec((D, D), lambda i: (0, 0)),
            pl.BlockSpec((D, 16), lambda i: (0, 0)),
            pl.BlockSpec((D, 16), lambda i: (0, 0)),
        ],
        out_specs=[
            pl.BlockSpec((rb, D), lambda i: (i, 0)),
            pl.BlockSpec((rb, 16), lambda i: (i, 0)),
            pl.BlockSpec((rb, 16), lambda i: (i, 0)),
        ],
        out_shape=[
            jax.ShapeDtypeStruct((2 * N_SRC, D), jnp.float32),
            jax.ShapeDtypeStruct((2 * N_SRC, 16), jnp.float32),
            jax.ShapeDtypeStruct((2 * N_SRC, 16), jnp.float32),
        ],
    )(x, w_gat, almat, armat)


def _sc_edge_body(h_hbm, el_hbm, er_hbm, src_hbm, dst_hbm, z128_hbm, z16_hbm,
                  num_out, den_out,
                  src_v, dst_v, el_v, er_v, w_v, h_v, num_s, den_s,
                  sem0, sem1, sem2):
    cid = lax.axis_index("c")
    sid = lax.axis_index("s")
    wid = sid * 2 + cid

    pltpu.sync_copy(z128_hbm.at[pl.ds(sid * ZR, ZR)],
                    num_s.at[pl.ds(sid * ZR, ZR)])
    pltpu.sync_copy(z16_hbm.at[pl.ds(sid * ZR, ZR)],
                    den_s.at[pl.ds(sid * ZR, ZR)])
    plsc.subcore_barrier()

    nchunks = EPW // CHUNK

    def chunk_body(c, carry):
        base = wid * EPW + c * CHUNK
        pltpu.sync_copy(src_hbm.at[pl.ds(base, CHUNK)], src_v)
        pltpu.sync_copy(dst_hbm.at[pl.ds(base, CHUNK)], dst_v)
        cp1 = pltpu.async_copy(h_hbm.at[src_v], h_v, sem0)
        cp2 = pltpu.async_copy(el_hbm.at[src_v], el_v, sem1)
        cp3 = pltpu.async_copy(er_hbm.at[dst_v], er_v, sem2)
        cp2.wait()
        cp3.wait()

        def edge_body(k, carry2):
            e = el_v[k] + er_v[k]
            e = jnp.where(e > 0, e, 0.2 * e)
            w_v[k] = jnp.exp(e)
            return carry2

        lax.fori_loop(0, CHUNK, edge_body, 0, unroll=4)
        cp1.wait()

        def msg_body(k, carry2):
            w = w_v[k]
            for hh in range(NH):
                blk = h_v[k, pl.ds(hh * HD, HD)]
                h_v[k, pl.ds(hh * HD, HD)] = blk * w[hh]
            return carry2

        lax.fori_loop(0, CHUNK, msg_body, 0)
        pltpu.sync_copy(h_v, num_s.at[dst_v], add=True)
        pltpu.sync_copy(w_v, den_s.at[dst_v], add=True)
        return carry

    lax.fori_loop(0, nchunks, chunk_body, 0)
    plsc.subcore_barrier()

    off = cid * N_ACC + sid * ZR
    pltpu.sync_copy(num_s.at[pl.ds(sid * ZR, ZR)], num_out.at[pl.ds(off, ZR)])
    pltpu.sync_copy(den_s.at[pl.ds(sid * ZR, ZR)], den_out.at[pl.ds(off, ZR)])


def _sc_edge(h_src, el_src, er_dst, src_idx, dst_idx, z128, z16):
    mesh = plsc.VectorSubcoreMesh(core_axis_name="c", subcore_axis_name="s")
    f = pl.kernel(
        _sc_edge_body,
        mesh=mesh,
        compiler_params=pltpu.CompilerParams(use_tc_tiling_on_sc=False),
        out_type=[
            jax.ShapeDtypeStruct((2 * N_ACC, D), jnp.float32),
            jax.ShapeDtypeStruct((2 * N_ACC, 16), jnp.float32),
        ],
        scratch_types=[
            pltpu.VMEM((CHUNK,), jnp.int32),
            pltpu.VMEM((CHUNK,), jnp.int32),
            pltpu.VMEM((CHUNK, 16), jnp.float32),
            pltpu.VMEM((CHUNK, 16), jnp.float32),
            pltpu.VMEM((CHUNK, 16), jnp.float32),
            pltpu.VMEM((CHUNK, D), jnp.float32),
            pltpu.VMEM_SHARED((N_ACC, D), jnp.float32),
            pltpu.VMEM_SHARED((N_ACC, 16), jnp.float32),
            pltpu.SemaphoreType.DMA,
            pltpu.SemaphoreType.DMA,
            pltpu.SemaphoreType.DMA,
        ],
    )
    return f(h_src, el_src, er_dst, src_idx, dst_idx, z128, z16)


def _ln(x, g, b):
    mu = jnp.mean(x, axis=-1, keepdims=True)
    var = jnp.mean((x - mu) ** 2, axis=-1, keepdims=True)
    return (x - mu) / jnp.sqrt(var + 1e-5) * g + b


def _fin_body(num0_ref, num1_ref, den0_ref, den1_ref, hd_ref, eld_ref,
              erd_ref, fd_ref,
              wvs_ref, bvs_ref, wos_ref, bos_ref,
              wq_ref, bq_ref, wk_ref, bk_ref, wv_ref, bv_ref, wo_ref, bo_ref,
              g1_ref, b1_ref, g2_ref, b2_ref, g3_ref, b3_ref,
              w1_ref, bf1_ref, w2_ref, bf2_ref,
              erep_ref, ered_ref,
              out_ref, gat_ref):
    erep = erep_ref[...]
    ered = ered_ref[...]

    eself = eld_ref[...] + erd_ref[...]
    eself = jnp.where(eself > 0, eself, 0.2 * eself)
    wself = jnp.exp(eself)
    wrep = jnp.dot(wself, erep, preferred_element_type=jnp.float32)
    den = jnp.dot(den0_ref[...] + den1_ref[...], erep,
                  preferred_element_type=jnp.float32) + wrep
    num = num0_ref[...] + num1_ref[...] + wrep * hd_ref[...]
    gat = num / den
    gat_ref[...] = gat

    t = fd_ref[...]
    sa = jnp.dot(jnp.dot(t, wvs_ref[...], preferred_element_type=jnp.float32),
                 wos_ref[...], preferred_element_type=jnp.float32)
    sa = sa + bvs_ref[...] @ wos_ref[...] + bos_ref[...]
    x = _ln(t + sa, g1_ref[...], b1_ref[...])

    q = jnp.dot(x, wq_ref[...], preferred_element_type=jnp.float32) + bq_ref[...]
    k1 = jnp.dot(t, wk_ref[...], preferred_element_type=jnp.float32) + bk_ref[...]
    k2 = jnp.dot(gat, wk_ref[...], preferred_element_type=jnp.float32) + bk_ref[...]
    v1 = jnp.dot(t, wv_ref[...], preferred_element_type=jnp.float32) + bv_ref[...]
    v2 = jnp.dot(gat, wv_ref[...], preferred_element_type=jnp.float32) + bv_ref[...]
    s1 = jnp.dot(q * k1, ered, preferred_element_type=jnp.float32)
    s2 = jnp.dot(q * k2, ered, preferred_element_type=jnp.float32)
    a1 = 1.0 / (1.0 + jnp.exp(s2 - s1))
    a1r = jnp.dot(a1, erep, preferred_element_type=jnp.float32)
    o = v2 + a1r * (v1 - v2)
    ca = jnp.dot(o, wo_ref[...], preferred_element_type=jnp.float32) + bo_ref[...]
    x = _ln(x + ca, g2_ref[...], b2_ref[...])

    hid = jnp.maximum(
        jnp.dot(x, w1_ref[...], preferred_element_type=jnp.float32)
        + bf1_ref[...], 0.0)
    ff = jnp.dot(hid, w2_ref[...], preferred_element_type=jnp.float32) + bf2_ref[...]
    out_ref[...] = _ln(x + ff, g3_ref[...], b3_ref[...])


def _finalize(num0, num1, den0, den1, h_dst, el_dst, er_dst, feat_dst,
              weights):
    nb = 25
    B = N_DST // nb

    def row(shape):
        return pl.BlockSpec(shape, lambda i: (i, 0))

    def full(shape):
        return pl.BlockSpec(shape, lambda i: (0, 0))

    in_specs = [
        row((B, D)), row((B, D)), row((B, 16)), row((B, 16)),
        row((B, D)), row((B, 16)), row((B, 16)), row((B, D)),
    ]
    weight_shapes = [w.shape for w in weights]
    in_specs += [full(s) for s in weight_shapes]
    return pl.pallas_call(
        _fin_body,
        grid=(nb,),
        in_specs=in_specs,
        out_specs=[row((B, D)), row((B, D))],
        out_shape=[
            jax.ShapeDtypeStruct((N_DST, D), jnp.float32),
            jax.ShapeDtypeStruct((N_DST, D), jnp.float32),
        ],
    )(num0, num1, den0, den1, h_dst, el_dst, er_dst, feat_dst, *weights)


def kernel(feat_src, feat_dst, params, edge_index):
    x = jnp.concatenate([feat_src, feat_dst], axis=0)
    rows = jnp.arange(D)
    colmask = (rows[:, None] // HD) == jnp.arange(16)[None, :]
    almat = jnp.where(colmask, params["attn_l"].reshape(-1)[:, None], 0.0)
    armat = jnp.where(colmask, params["attn_r"].reshape(-1)[:, None], 0.0)
    erep = colmask.astype(jnp.float32).T
    ered = colmask.astype(jnp.float32) * 0.25

    pad = E_PAD - E
    src_p = jnp.concatenate(
        [edge_index[0].astype(jnp.int32), jnp.zeros((pad,), jnp.int32)])
    dst_p = jnp.concatenate(
        [edge_index[1].astype(jnp.int32),
         jnp.full((pad,), N_DST, jnp.int32)])
    z128 = jnp.zeros((N_ACC, D), jnp.float32)
    z16 = jnp.zeros((N_ACC, 16), jnp.float32)

    h_all, el16, er16 = _gat_pre(x, params["W_gat"], almat, armat)
    h_src = h_all[:N_SRC]
    h_dst = h_all[N_SRC:]
    el_src = el16[:N_SRC]
    el_dst = el16[N_SRC:]
    er_dst = er16[N_SRC:]

    num_parts, den_parts = _sc_edge(h_src, el_src, er_dst, src_p, dst_p,
                                    z128, z16)
    num0 = num_parts[:N_DST]
    num1 = num_parts[N_ACC:N_ACC + N_DST]
    den0 = den_parts[:N_DST, :]
    den1 = den_parts[N_ACC:N_ACC + N_DST, :]

    sa_p, ca_p = params["sa"], params["ca"]

    def r1(v):
        return v.reshape(1, -1)

    weights = [
        sa_p["Wv"], r1(sa_p["bv"]), sa_p["Wo"], r1(sa_p["bo"]),
        ca_p["Wq"], r1(ca_p["bq"]), ca_p["Wk"], r1(ca_p["bk"]),
        ca_p["Wv"], r1(ca_p["bv"]), ca_p["Wo"], r1(ca_p["bo"]),
        r1(params["ln1_g"]), r1(params["ln1_b"]),
        r1(params["ln2_g"]), r1(params["ln2_b"]),
        r1(params["ln3_g"]), r1(params["ln3_b"]),
        params["W1"], r1(params["b1"]), params["W2"], r1(params["b2"]),
        erep, ered,
    ]
    out, gat = _finalize(num0, num1, den0, den1, h_dst, el_dst, er_dst,
                         feat_dst, weights)
    return out, gat


# speedup vs baseline: 51.1638x; 1.0858x over previous
import functools

import jax
import jax.numpy as jnp
from jax import lax
from jax.experimental import pallas as pl
from jax.experimental.pallas import tpu as pltpu
from jax.experimental.pallas import tpu_sc as plsc

D = 128
NH = 8
HD = 16
N_SRC = 10000
N_DST = 10000
E = 320000
FF = 2048

NW = 32
CHUNK = 64
EPW = 10240
E_PAD = NW * EPW
N_ACC = 10112
ZR = N_ACC // 16


def _pre_body(x_ref, w_ref, al_ref, ar_ref, h_ref, el_ref, er_ref):
    h = jnp.dot(x_ref[...], w_ref[...], preferred_element_type=jnp.float32)
    h_ref[...] = h
    el_ref[...] = jnp.dot(h, al_ref[...], preferred_element_type=jnp.float32)
    er_ref[...] = jnp.dot(h, ar_ref[...], preferred_element_type=jnp.float32)


def _gat_pre(x, w_gat, almat, armat):
    nb = 10
    rb = (2 * N_SRC) // nb
    return pl.pallas_call(
        _pre_body,
        grid=(nb,),
        in_specs=[
            pl.BlockSpec((rb, D), lambda i: (i, 0)),
            pl.BlockSpec((D, D), lambda i: (0, 0)),
            pl.BlockSpec((D, 16), lambda i: (0, 0)),
            pl.BlockSpec((D, 16), lambda i: (0, 0)),
        ],
        out_specs=[
            pl.BlockSpec((rb, D), lambda i: (i, 0)),
            pl.BlockSpec((rb, 16), lambda i: (i, 0)),
            pl.BlockSpec((rb, 16), lambda i: (i, 0)),
        ],
        out_shape=[
            jax.ShapeDtypeStruct((2 * N_SRC, D), jnp.float32),
            jax.ShapeDtypeStruct((2 * N_SRC, 16), jnp.float32),
            jax.ShapeDtypeStruct((2 * N_SRC, 16), jnp.float32),
        ],
    )(x, w_gat, almat, armat)


def _sc_edge_body(h_hbm, el_hbm, er_hbm, src_hbm, dst_hbm, z128_hbm, z16_hbm,
                  num_out, den_out,
                  src_v, dst_v, sdst_v, el_v, er_v, w_v, h_v, msg_v,
                  num_s, den_s,
                  gsem0, gsem1, nsem0, nsem1, dsem0, dsem1):
    cid = lax.axis_index("c")
    sid = lax.axis_index("s")
    wid = sid * 2 + cid
    gsem = (gsem0, gsem1)
    nsem = (nsem0, nsem1)
    dsem = (dsem0, dsem1)

    pltpu.sync_copy(z128_hbm.at[pl.ds(sid * ZR, ZR)],
                    num_s.at[pl.ds(sid * ZR, ZR)])
    pltpu.sync_copy(z16_hbm.at[pl.ds(sid * ZR, ZR)],
                    den_s.at[pl.ds(sid * ZR, ZR)])
    plsc.subcore_barrier()

    nchunks = EPW // CHUNK

    def issue_gathers(c, b):
        base = wid * EPW + c * CHUNK
        pltpu.sync_copy(src_hbm.at[pl.ds(base, CHUNK)], src_v.at[b])
        pltpu.sync_copy(dst_hbm.at[pl.ds(base, CHUNK)], dst_v.at[b])
        pltpu.async_copy(h_hbm.at[src_v.at[b]], h_v.at[b], gsem[b])
        pltpu.async_copy(el_hbm.at[src_v.at[b]], el_v.at[b], gsem[b])
        pltpu.async_copy(er_hbm.at[dst_v.at[b]], er_v.at[b], gsem[b])

    def wait_gathers(b):
        pltpu.make_async_copy(h_hbm.at[src_v.at[b]], h_v.at[b],
                              gsem[b]).wait()
        pltpu.make_async_copy(el_hbm.at[src_v.at[b]], el_v.at[b],
                              gsem[b]).wait()
        pltpu.make_async_copy(er_hbm.at[dst_v.at[b]], er_v.at[b],
                              gsem[b]).wait()

    def wait_scatters(b):
        pltpu.make_async_copy(msg_v.at[b], num_s.at[sdst_v.at[b]],
                              nsem[b]).wait()
        pltpu.make_async_copy(w_v.at[b], den_s.at[sdst_v.at[b]],
                              dsem[b]).wait()

    def compute(b):
        hb, mb = h_v.at[b], msg_v.at[b]
        elb, erb, wb = el_v.at[b], er_v.at[b], w_v.at[b]

        def edge(k, carry):
            e = elb[k] + erb[k]
            e = jnp.where(e > 0, e, 0.2 * e)
            w = jnp.exp(e)
            wb[k] = w
            for hh in range(NH):
                mb[k, pl.ds(hh * HD, HD)] = hb[k, pl.ds(hh * HD, HD)] * w[hh]
            return carry

        lax.fori_loop(0, CHUNK, edge, 0, unroll=2)

    issue_gathers(0, 0)
    issue_gathers(1, 1)

    def pair(p, carry):
        for b in range(2):
            c = 2 * p + b
            wait_gathers(b)

            @pl.when(p > 0)
            def _():
                wait_scatters(b)

            compute(b)
            for i in range(CHUNK // 16):
                sdst_v.at[b][pl.ds(i * 16, 16)] = dst_v.at[b][pl.ds(i * 16, 16)]
            pltpu.async_copy(msg_v.at[b], num_s.at[sdst_v.at[b]], nsem[b],
                             add=True)
            pltpu.async_copy(w_v.at[b], den_s.at[sdst_v.at[b]], dsem[b],
                             add=True)

            @pl.when(c + 2 < nchunks)
            def _():
                issue_gathers(c + 2, b)
        return carry

    lax.fori_loop(0, nchunks // 2, pair, 0)
    for b in range(2):
        wait_scatters(b)
    plsc.subcore_barrier()

    off = cid * N_ACC + sid * ZR
    pltpu.sync_copy(num_s.at[pl.ds(sid * ZR, ZR)], num_out.at[pl.ds(off, ZR)])
    pltpu.sync_copy(den_s.at[pl.ds(sid * ZR, ZR)], den_out.at[pl.ds(off, ZR)])


def _sc_edge(h_src, el_src, er_dst, src_idx, dst_idx, z128, z16):
    mesh = plsc.VectorSubcoreMesh(core_axis_name="c", subcore_axis_name="s")
    f = pl.kernel(
        _sc_edge_body,
        mesh=mesh,
        compiler_params=pltpu.CompilerParams(use_tc_tiling_on_sc=False),
        out_type=[
            jax.ShapeDtypeStruct((2 * N_ACC, D), jnp.float32),
            jax.ShapeDtypeStruct((2 * N_ACC, 16), jnp.float32),
        ],
        scratch_types=[
            pltpu.VMEM((2, CHUNK), jnp.int32),
            pltpu.VMEM((2, CHUNK), jnp.int32),
            pltpu.VMEM((2, CHUNK), jnp.int32),
            pltpu.VMEM((2, CHUNK, 16), jnp.float32),
            pltpu.VMEM((2, CHUNK, 16), jnp.float32),
            pltpu.VMEM((2, CHUNK, 16), jnp.float32),
            pltpu.VMEM((2, CHUNK, D), jnp.float32),
            pltpu.VMEM((2, CHUNK, D), jnp.float32),
            pltpu.VMEM_SHARED((N_ACC, D), jnp.float32),
            pltpu.VMEM_SHARED((N_ACC, 16), jnp.float32),
            pltpu.SemaphoreType.DMA,
            pltpu.SemaphoreType.DMA,
            pltpu.SemaphoreType.DMA,
            pltpu.SemaphoreType.DMA,
            pltpu.SemaphoreType.DMA,
            pltpu.SemaphoreType.DMA,
        ],
    )
    return f(h_src, el_src, er_dst, src_idx, dst_idx, z128, z16)


def _ln(x, g, b):
    mu = jnp.mean(x, axis=-1, keepdims=True)
    var = jnp.mean((x - mu) ** 2, axis=-1, keepdims=True)
    return (x - mu) / jnp.sqrt(var + 1e-5) * g + b


def _fin_body(num0_ref, num1_ref, den0_ref, den1_ref, hd_ref, eld_ref,
              erd_ref, fd_ref,
              wvs_ref, bvs_ref, wos_ref, bos_ref,
              wq_ref, bq_ref, wk_ref, bk_ref, wv_ref, bv_ref, wo_ref, bo_ref,
              g1_ref, b1_ref, g2_ref, b2_ref, g3_ref, b3_ref,
              w1_ref, bf1_ref, w2_ref, bf2_ref,
              erep_ref, ered_ref,
              out_ref, gat_ref):
    erep = erep_ref[...]
    ered = ered_ref[...]

    eself = eld_ref[...] + erd_ref[...]
    eself = jnp.where(eself > 0, eself, 0.2 * eself)
    wself = jnp.exp(eself)
    wrep = jnp.dot(wself, erep, preferred_element_type=jnp.float32)
    den = jnp.dot(den0_ref[...] + den1_ref[...], erep,
                  preferred_element_type=jnp.float32) + wrep
    num = num0_ref[...] + num1_ref[...] + wrep * hd_ref[...]
    gat = num / den
    gat_ref[...] = gat

    t = fd_ref[...]
    sa = jnp.dot(jnp.dot(t, wvs_ref[...], preferred_element_type=jnp.float32),
                 wos_ref[...], preferred_element_type=jnp.float32)
    sa = sa + bvs_ref[...] @ wos_ref[...] + bos_ref[...]
    x = _ln(t + sa, g1_ref[...], b1_ref[...])

    q = jnp.dot(x, wq_ref[...], preferred_element_type=jnp.float32) + bq_ref[...]
    k1 = jnp.dot(t, wk_ref[...], preferred_element_type=jnp.float32) + bk_ref[...]
    k2 = jnp.dot(gat, wk_ref[...], preferred_element_type=jnp.float32) + bk_ref[...]
    v1 = jnp.dot(t, wv_ref[...], preferred_element_type=jnp.float32) + bv_ref[...]
    v2 = jnp.dot(gat, wv_ref[...], preferred_element_type=jnp.float32) + bv_ref[...]
    s1 = jnp.dot(q * k1, ered, preferred_element_type=jnp.float32)
    s2 = jnp.dot(q * k2, ered, preferred_element_type=jnp.float32)
    a1 = 1.0 / (1.0 + jnp.exp(s2 - s1))
    a1r = jnp.dot(a1, erep, preferred_element_type=jnp.float32)
    o = v2 + a1r * (v1 - v2)
    ca = jnp.dot(o, wo_ref[...], preferred_element_type=jnp.float32) + bo_ref[...]
    x = _ln(x + ca, g2_ref[...], b2_ref[...])

    hid = jnp.maximum(
        jnp.dot(x, w1_ref[...], preferred_element_type=jnp.float32)
        + bf1_ref[...], 0.0)
    ff = jnp.dot(hid, w2_ref[...], preferred_element_type=jnp.float32) + bf2_ref[...]
    out_ref[...] = _ln(x + ff, g3_ref[...], b3_ref[...])


def _finalize(num0, num1, den0, den1, h_dst, el_dst, er_dst, feat_dst,
              weights):
    nb = 25
    B = N_DST // nb

    def row(shape):
        return pl.BlockSpec(shape, lambda i: (i, 0))

    def full(shape):
        return pl.BlockSpec(shape, lambda i: (0, 0))

    in_specs = [
        row((B, D)), row((B, D)), row((B, 16)), row((B, 16)),
        row((B, D)), row((B, 16)), row((B, 16)), row((B, D)),
    ]
    weight_shapes = [w.shape for w in weights]
    in_specs += [full(s) for s in weight_shapes]
    return pl.pallas_call(
        _fin_body,
        grid=(nb,),
        in_specs=in_specs,
        out_specs=[row((B, D)), row((B, D))],
        out_shape=[
            jax.ShapeDtypeStruct((N_DST, D), jnp.float32),
            jax.ShapeDtypeStruct((N_DST, D), jnp.float32),
        ],
    )(num0, num1, den0, den1, h_dst, el_dst, er_dst, feat_dst, *weights)


def kernel(feat_src, feat_dst, params, edge_index):
    x = jnp.concatenate([feat_src, feat_dst], axis=0)
    rows = jnp.arange(D)
    colmask = (rows[:, None] // HD) == jnp.arange(16)[None, :]
    almat = jnp.where(colmask, params["attn_l"].reshape(-1)[:, None], 0.0)
    armat = jnp.where(colmask, params["attn_r"].reshape(-1)[:, None], 0.0)
    erep = colmask.astype(jnp.float32).T
    ered = colmask.astype(jnp.float32) * 0.25

    pad = E_PAD - E
    src_p = jnp.concatenate(
        [edge_index[0].astype(jnp.int32), jnp.zeros((pad,), jnp.int32)])
    dst_p = jnp.concatenate(
        [edge_index[1].astype(jnp.int32),
         jnp.full((pad,), N_DST, jnp.int32)])
    z128 = jnp.zeros((N_ACC, D), jnp.float32)
    z16 = jnp.zeros((N_ACC, 16), jnp.float32)

    h_all, el16, er16 = _gat_pre(x, params["W_gat"], almat, armat)
    h_src = h_all[:N_SRC]
    h_dst = h_all[N_SRC:]
    el_src = el16[:N_SRC]
    el_dst = el16[N_SRC:]
    er_dst = er16[N_SRC:]

    num_parts, den_parts = _sc_edge(h_src, el_src, er_dst, src_p, dst_p,
                                    z128, z16)
    num0 = num_parts[:N_DST]
    num1 = num_parts[N_ACC:N_ACC + N_DST]
    den0 = den_parts[:N_DST, :]
    den1 = den_parts[N_ACC:N_ACC + N_DST, :]

    sa_p, ca_p = params["sa"], params["ca"]

    def r1(v):
        return v.reshape(1, -1)

    weights = [
        sa_p["Wv"], r1(sa_p["bv"]), sa_p["Wo"], r1(sa_p["bo"]),
        ca_p["Wq"], r1(ca_p["bq"]), ca_p["Wk"], r1(ca_p["bk"]),
        ca_p["Wv"], r1(ca_p["bv"]), ca_p["Wo"], r1(ca_p["bo"]),
        r1(params["ln1_g"]), r1(params["ln1_b"]),
        r1(params["ln2_g"]), r1(params["ln2_b"]),
        r1(params["ln3_g"]), r1(params["ln3_b"]),
        params["W1"], r1(params["b1"]), params["W2"], r1(params["b2"]),
        erep, ered,
    ]
    out, gat = _finalize(num0, num1, den0, den1, h_dst, el_dst, er_dst,
                         feat_dst, weights)
    return out, gat


# speedup vs baseline: 61.4686x; 1.2014x over previous
import functools

import jax
import jax.numpy as jnp
from jax import lax
from jax.experimental import pallas as pl
from jax.experimental.pallas import tpu as pltpu
from jax.experimental.pallas import tpu_sc as plsc

D = 128
NH = 8
HD = 16
N_SRC = 10000
N_DST = 10000
E = 320000
FF = 2048

NW = 32
CHUNK = 64
EPW = 10240
E_PAD = NW * EPW
N_ACC = 10112
ZR = N_ACC // 16


def _pre_body(x_ref, w_ref, al_ref, ar_ref, pm_ref,
              h_ref, hb_ref, el_ref, er_ref):
    h = jnp.dot(x_ref[...], w_ref[...], preferred_element_type=jnp.float32)
    h_ref[...] = h
    hb_ref[...] = jnp.dot(h, pm_ref[...],
                          preferred_element_type=jnp.float32).astype(
                              jnp.bfloat16)
    el_ref[...] = jnp.dot(h, al_ref[...], preferred_element_type=jnp.float32)
    er_ref[...] = jnp.dot(h, ar_ref[...], preferred_element_type=jnp.float32)


def _gat_pre(x, w_gat, almat, armat, permat):
    nb = 10
    rb = (2 * N_SRC) // nb
    return pl.pallas_call(
        _pre_body,
        grid=(nb,),
        in_specs=[
            pl.BlockSpec((rb, D), lambda i: (i, 0)),
            pl.BlockSpec((D, D), lambda i: (0, 0)),
            pl.BlockSpec((D, 16), lambda i: (0, 0)),
            pl.BlockSpec((D, 16), lambda i: (0, 0)),
            pl.BlockSpec((D, D), lambda i: (0, 0)),
        ],
        out_specs=[
            pl.BlockSpec((rb, D), lambda i: (i, 0)),
            pl.BlockSpec((rb, D), lambda i: (i, 0)),
            pl.BlockSpec((rb, 16), lambda i: (i, 0)),
            pl.BlockSpec((rb, 16), lambda i: (i, 0)),
        ],
        out_shape=[
            jax.ShapeDtypeStruct((2 * N_SRC, D), jnp.float32),
            jax.ShapeDtypeStruct((2 * N_SRC, D), jnp.bfloat16),
            jax.ShapeDtypeStruct((2 * N_SRC, 16), jnp.float32),
            jax.ShapeDtypeStruct((2 * N_SRC, 16), jnp.float32),
        ],
    )(x, w_gat, almat, armat, permat)


def _sc_edge_body(h_hbm, el_hbm, er_hbm, src_hbm, dst_hbm, z128_hbm, z16_hbm,
                  num_out, den_out,
                  src_v, dst_v, sdst_v, el_v, er_v, w_v, h_v, msg_v,
                  num_s, den_s,
                  gsem0, gsem1, nsem0, nsem1, dsem0, dsem1):
    cid = lax.axis_index("c")
    sid = lax.axis_index("s")
    wid = sid * 2 + cid
    gsem = (gsem0, gsem1)
    nsem = (nsem0, nsem1)
    dsem = (dsem0, dsem1)

    pltpu.sync_copy(z128_hbm.at[pl.ds(sid * ZR, ZR)],
                    num_s.at[pl.ds(sid * ZR, ZR)])
    pltpu.sync_copy(z16_hbm.at[pl.ds(sid * ZR, ZR)],
                    den_s.at[pl.ds(sid * ZR, ZR)])
    plsc.subcore_barrier()

    nchunks = EPW // CHUNK

    def issue_gathers(c, b):
        base = wid * EPW + c * CHUNK
        pltpu.sync_copy(src_hbm.at[pl.ds(base, CHUNK)], src_v.at[b])
        pltpu.sync_copy(dst_hbm.at[pl.ds(base, CHUNK)], dst_v.at[b])
        pltpu.async_copy(h_hbm.at[src_v.at[b]], h_v.at[b], gsem[b])
        pltpu.async_copy(el_hbm.at[src_v.at[b]], el_v.at[b], gsem[b])
        pltpu.async_copy(er_hbm.at[dst_v.at[b]], er_v.at[b], gsem[b])

    def wait_gathers(b):
        pltpu.make_async_copy(h_hbm.at[src_v.at[b]], h_v.at[b],
                              gsem[b]).wait()
        pltpu.make_async_copy(el_hbm.at[src_v.at[b]], el_v.at[b],
                              gsem[b]).wait()
        pltpu.make_async_copy(er_hbm.at[dst_v.at[b]], er_v.at[b],
                              gsem[b]).wait()

    def wait_scatters(b):
        pltpu.make_async_copy(msg_v.at[b], num_s.at[sdst_v.at[b]],
                              nsem[b]).wait()
        pltpu.make_async_copy(w_v.at[b], den_s.at[sdst_v.at[b]],
                              dsem[b]).wait()

    def compute(b):
        hb, mb = h_v.at[b], msg_v.at[b]
        elb, erb, wb = el_v.at[b], er_v.at[b], w_v.at[b]

        def edge(k, carry):
            e = elb[k] + erb[k]
            e = jnp.where(e > 0, e, 0.2 * e)
            w = jnp.exp(e)
            wb[k] = w
            for q in range(NH // 2):
                ha, hb2 = plsc.unpack(hb[k, pl.ds(q * 32, 32)],
                                      format=plsc.PackFormat.INTERLEAVED)
                mb[k, pl.ds((2 * q) * HD, HD)] = ha * w[2 * q]
                mb[k, pl.ds((2 * q + 1) * HD, HD)] = hb2 * w[2 * q + 1]
            return carry

        lax.fori_loop(0, CHUNK, edge, 0, unroll=2)

    issue_gathers(0, 0)
    issue_gathers(1, 1)

    def pair(p, carry):
        for b in range(2):
            c = 2 * p + b
            wait_gathers(b)

            @pl.when(p > 0)
            def _():
                wait_scatters(b)

            compute(b)
            for i in range(CHUNK // 16):
                sdst_v.at[b][pl.ds(i * 16, 16)] = dst_v.at[b][pl.ds(i * 16, 16)]
            pltpu.async_copy(msg_v.at[b], num_s.at[sdst_v.at[b]], nsem[b],
                             add=True)
            pltpu.async_copy(w_v.at[b], den_s.at[sdst_v.at[b]], dsem[b],
                             add=True)

            @pl.when(c + 2 < nchunks)
            def _():
                issue_gathers(c + 2, b)
        return carry

    lax.fori_loop(0, nchunks // 2, pair, 0)
    for b in range(2):
        wait_scatters(b)
    plsc.subcore_barrier()

    off = cid * N_ACC + sid * ZR
    pltpu.sync_copy(num_s.at[pl.ds(sid * ZR, ZR)], num_out.at[pl.ds(off, ZR)])
    pltpu.sync_copy(den_s.at[pl.ds(sid * ZR, ZR)], den_out.at[pl.ds(off, ZR)])


def _sc_edge(h_src, el_src, er_dst, src_idx, dst_idx, z128, z16):
    mesh = plsc.VectorSubcoreMesh(core_axis_name="c", subcore_axis_name="s")
    f = pl.kernel(
        _sc_edge_body,
        mesh=mesh,
        compiler_params=pltpu.CompilerParams(use_tc_tiling_on_sc=False,
                                             needs_layout_passes=False),
        out_type=[
            jax.ShapeDtypeStruct((2 * N_ACC, D), jnp.float32),
            jax.ShapeDtypeStruct((2 * N_ACC, 16), jnp.float32),
        ],
        scratch_types=[
            pltpu.VMEM((2, CHUNK), jnp.int32),
            pltpu.VMEM((2, CHUNK), jnp.int32),
            pltpu.VMEM((2, CHUNK), jnp.int32),
            pltpu.VMEM((2, CHUNK, 16), jnp.float32),
            pltpu.VMEM((2, CHUNK, 16), jnp.float32),
            pltpu.VMEM((2, CHUNK, 16), jnp.float32),
            pltpu.VMEM((2, CHUNK, D), jnp.bfloat16),
            pltpu.VMEM((2, CHUNK, D), jnp.float32),
            pltpu.VMEM_SHARED((N_ACC, D), jnp.float32),
            pltpu.VMEM_SHARED((N_ACC, 16), jnp.float32),
            pltpu.SemaphoreType.DMA,
            pltpu.SemaphoreType.DMA,
            pltpu.SemaphoreType.DMA,
            pltpu.SemaphoreType.DMA,
            pltpu.SemaphoreType.DMA,
            pltpu.SemaphoreType.DMA,
        ],
    )
    return f(h_src, el_src, er_dst, src_idx, dst_idx, z128, z16)


def _ln(x, g, b):
    mu = jnp.mean(x, axis=-1, keepdims=True)
    var = jnp.mean((x - mu) ** 2, axis=-1, keepdims=True)
    return (x - mu) / jnp.sqrt(var + 1e-5) * g + b


def _fin_body(num0_ref, num1_ref, den0_ref, den1_ref, hd_ref, eld_ref,
              erd_ref, fd_ref,
              wvs_ref, bvs_ref, wos_ref, bos_ref,
              wq_ref, bq_ref, wk_ref, bk_ref, wv_ref, bv_ref, wo_ref, bo_ref,
              g1_ref, b1_ref, g2_ref, b2_ref, g3_ref, b3_ref,
              w1_ref, bf1_ref, w2_ref, bf2_ref,
              erep_ref, ered_ref,
              out_ref, gat_ref):
    erep = erep_ref[...]
    ered = ered_ref[...]

    eself = eld_ref[...] + erd_ref[...]
    eself = jnp.where(eself > 0, eself, 0.2 * eself)
    wself = jnp.exp(eself)
    wrep = jnp.dot(wself, erep, preferred_element_type=jnp.float32)
    den = jnp.dot(den0_ref[...] + den1_ref[...], erep,
                  preferred_element_type=jnp.float32) + wrep
    num = num0_ref[...] + num1_ref[...] + wrep * hd_ref[...]
    gat = num / den
    gat_ref[...] = gat

    t = fd_ref[...]
    sa = jnp.dot(jnp.dot(t, wvs_ref[...], preferred_element_type=jnp.float32),
                 wos_ref[...], preferred_element_type=jnp.float32)
    sa = sa + bvs_ref[...] @ wos_ref[...] + bos_ref[...]
    x = _ln(t + sa, g1_ref[...], b1_ref[...])

    q = jnp.dot(x, wq_ref[...], preferred_element_type=jnp.float32) + bq_ref[...]
    k1 = jnp.dot(t, wk_ref[...], preferred_element_type=jnp.float32) + bk_ref[...]
    k2 = jnp.dot(gat, wk_ref[...], preferred_element_type=jnp.float32) + bk_ref[...]
    v1 = jnp.dot(t, wv_ref[...], preferred_element_type=jnp.float32) + bv_ref[...]
    v2 = jnp.dot(gat, wv_ref[...], preferred_element_type=jnp.float32) + bv_ref[...]
    s1 = jnp.dot(q * k1, ered, preferred_element_type=jnp.float32)
    s2 = jnp.dot(q * k2, ered, preferred_element_type=jnp.float32)
    a1 = 1.0 / (1.0 + jnp.exp(s2 - s1))
    a1r = jnp.dot(a1, erep, preferred_element_type=jnp.float32)
    o = v2 + a1r * (v1 - v2)
    ca = jnp.dot(o, wo_ref[...], preferred_element_type=jnp.float32) + bo_ref[...]
    x = _ln(x + ca, g2_ref[...], b2_ref[...])

    hid = jnp.maximum(
        jnp.dot(x, w1_ref[...], preferred_element_type=jnp.float32)
        + bf1_ref[...], 0.0)
    ff = jnp.dot(hid, w2_ref[...], preferred_element_type=jnp.float32) + bf2_ref[...]
    out_ref[...] = _ln(x + ff, g3_ref[...], b3_ref[...])


def _finalize(num0, num1, den0, den1, h_dst, el_dst, er_dst, feat_dst,
              weights):
    nb = 25
    B = N_DST // nb

    def row(shape):
        return pl.BlockSpec(shape, lambda i: (i, 0))

    def full(shape):
        return pl.BlockSpec(shape, lambda i: (0, 0))

    in_specs = [
        row((B, D)), row((B, D)), row((B, 16)), row((B, 16)),
        row((B, D)), row((B, 16)), row((B, 16)), row((B, D)),
    ]
    weight_shapes = [w.shape for w in weights]
    in_specs += [full(s) for s in weight_shapes]
    return pl.pallas_call(
        _fin_body,
        grid=(nb,),
        in_specs=in_specs,
        out_specs=[row((B, D)), row((B, D))],
        out_shape=[
            jax.ShapeDtypeStruct((N_DST, D), jnp.float32),
            jax.ShapeDtypeStruct((N_DST, D), jnp.float32),
        ],
    )(num0, num1, den0, den1, h_dst, el_dst, er_dst, feat_dst, *weights)


def kernel(feat_src, feat_dst, params, edge_index):
    x = jnp.concatenate([feat_src, feat_dst], axis=0)
    rows = jnp.arange(D)
    colmask = (rows[:, None] // HD) == jnp.arange(16)[None, :]
    almat = jnp.where(colmask, params["attn_l"].reshape(-1)[:, None], 0.0)
    armat = jnp.where(colmask, params["attn_r"].reshape(-1)[:, None], 0.0)
    erep = colmask.astype(jnp.float32).T
    ered = colmask.astype(jnp.float32) * 0.25

    wv = rows // 32
    rem = rows % 32
    half = rem // 16
    ii = rem % 16
    pcol = 32 * wv + 2 * ii + half
    permat = (pcol[:, None] == rows[None, :]).astype(jnp.float32)

    pad = E_PAD - E
    src_p = jnp.concatenate(
        [edge_index[0].astype(jnp.int32), jnp.zeros((pad,), jnp.int32)])
    dst_p = jnp.concatenate(
        [edge_index[1].astype(jnp.int32),
         jnp.full((pad,), N_DST, jnp.int32)])
    z128 = jnp.zeros((N_ACC, D), jnp.float32)
    z16 = jnp.zeros((N_ACC, 16), jnp.float32)

    h_all, hb_all, el16, er16 = _gat_pre(x, params["W_gat"], almat, armat,
                                         permat)
    hb_src = hb_all[:N_SRC]
    h_dst = h_all[N_SRC:]
    el_src = el16[:N_SRC]
    el_dst = el16[N_SRC:]
    er_dst = er16[N_SRC:]

    num_parts, den_parts = _sc_edge(hb_src, el_src, er_dst, src_p, dst_p,
                                    z128, z16)
    num0 = num_parts[:N_DST]
    num1 = num_parts[N_ACC:N_ACC + N_DST]
    den0 = den_parts[:N_DST, :]
    den1 = den_parts[N_ACC:N_ACC + N_DST, :]

    sa_p, ca_p = params["sa"], params["ca"]

    def r1(v):
        return v.reshape(1, -1)

    weights = [
        sa_p["Wv"], r1(sa_p["bv"]), sa_p["Wo"], r1(sa_p["bo"]),
        ca_p["Wq"], r1(ca_p["bq"]), ca_p["Wk"], r1(ca_p["bk"]),
        ca_p["Wv"], r1(ca_p["bv"]), ca_p["Wo"], r1(ca_p["bo"]),
        r1(params["ln1_g"]), r1(params["ln1_b"]),
        r1(params["ln2_g"]), r1(params["ln2_b"]),
        r1(params["ln3_g"]), r1(params["ln3_b"]),
        params["W1"], r1(params["b1"]), params["W2"], r1(params["b2"]),
        erep, ered,
    ]
    out, gat = _finalize(num0, num1, den0, den1, h_dst, el_dst, er_dst,
                         feat_dst, weights)
    return out, gat


# speedup vs baseline: 62.5851x; 1.0182x over previous
import functools

import jax
import jax.numpy as jnp
from jax import lax
from jax.experimental import pallas as pl
from jax.experimental.pallas import tpu as pltpu
from jax.experimental.pallas import tpu_sc as plsc

D = 128
NH = 8
HD = 16
N_SRC = 10000
N_DST = 10000
E = 320000
FF = 2048

NW = 32
CHUNK = 64
EPW = 10240
E_PAD = NW * EPW
N_ACC = 10112
ZR = N_ACC // 16
AW = D + HD


def _asrc_body(x_ref, w_ref, al32_ref, pm_ref, hbel_ref):
    h = jnp.dot(x_ref[...], w_ref[...], preferred_element_type=jnp.float32)
    hp = jnp.dot(h, pm_ref[...], preferred_element_type=jnp.float32)
    elp = jnp.dot(h, al32_ref[...], preferred_element_type=jnp.float32)
    hbel_ref[...] = jnp.concatenate(
        [hp.astype(jnp.bfloat16), elp.astype(jnp.bfloat16)], axis=-1)


def _gat_pre_src(x, w_gat, al32, permat):
    nb = 5
    rb = N_SRC // nb
    return pl.pallas_call(
        _asrc_body,
        grid=(nb,),
        in_specs=[
            pl.BlockSpec((rb, D), lambda i: (i, 0)),
            pl.BlockSpec((D, D), lambda i: (0, 0)),
            pl.BlockSpec((D, 32), lambda i: (0, 0)),
            pl.BlockSpec((D, D), lambda i: (0, 0)),
        ],
        out_specs=pl.BlockSpec((rb, D + 32), lambda i: (i, 0)),
        out_shape=jax.ShapeDtypeStruct((N_SRC, D + 32), jnp.bfloat16),
    )(x, w_gat, al32, permat)


def _adst_body(x_ref, w_ref, al_ref, ar_ref, h_ref, el_ref, er_ref):
    h = jnp.dot(x_ref[...], w_ref[...], preferred_element_type=jnp.float32)
    h_ref[...] = h
    el_ref[...] = jnp.dot(h, al_ref[...], preferred_element_type=jnp.float32)
    er_ref[...] = jnp.dot(h, ar_ref[...], preferred_element_type=jnp.float32)


def _gat_pre_dst(x, w_gat, almat, armat):
    nb = 5
    rb = N_DST // nb
    return pl.pallas_call(
        _adst_body,
        grid=(nb,),
        in_specs=[
            pl.BlockSpec((rb, D), lambda i: (i, 0)),
            pl.BlockSpec((D, D), lambda i: (0, 0)),
            pl.BlockSpec((D, 16), lambda i: (0, 0)),
            pl.BlockSpec((D, 16), lambda i: (0, 0)),
        ],
        out_specs=[
            pl.BlockSpec((rb, D), lambda i: (i, 0)),
            pl.BlockSpec((rb, 16), lambda i: (i, 0)),
            pl.BlockSpec((rb, 16), lambda i: (i, 0)),
        ],
        out_shape=[
            jax.ShapeDtypeStruct((N_DST, D), jnp.float32),
            jax.ShapeDtypeStruct((N_DST, 16), jnp.float32),
            jax.ShapeDtypeStruct((N_DST, 16), jnp.float32),
        ],
    )(x, w_gat, almat, armat)


def _sc_edge_body(hbel_hbm, er_hbm, src_hbm, dst_hbm,
                  acc0_out, acc1_out,
                  src_v, dst_v, sdst_v, er_v, hbel_v, msg_v,
                  acc_s,
                  gsem0, gsem1, ssem0, ssem1):
    cid = lax.axis_index("c")
    sid = lax.axis_index("s")
    wid = sid * 2 + cid
    gsem = (gsem0, gsem1)
    ssem = (ssem0, ssem1)

    zv = jnp.zeros((16,), jnp.float32)
    zb = msg_v.at[0]

    def zrow(r, carry):
        for j in range(AW // 16):
            zb[r, pl.ds(j * 16, 16)] = zv
        return carry

    lax.fori_loop(0, CHUNK, zrow, 0, unroll=4)
    base = sid * ZR
    for j in range(ZR // CHUNK):
        pltpu.sync_copy(zb, acc_s.at[pl.ds(base + j * CHUNK, CHUNK)])
    rem_rows = ZR % CHUNK
    if rem_rows:
        pltpu.sync_copy(zb.at[pl.ds(0, rem_rows)],
                        acc_s.at[pl.ds(base + (ZR // CHUNK) * CHUNK,
                                       rem_rows)])
    plsc.subcore_barrier()

    nchunks = EPW // CHUNK

    def issue_gathers(c, b):
        base = wid * EPW + c * CHUNK
        pltpu.sync_copy(src_hbm.at[pl.ds(base, CHUNK)], src_v.at[b])
        pltpu.sync_copy(dst_hbm.at[pl.ds(base, CHUNK)], dst_v.at[b])
        pltpu.async_copy(hbel_hbm.at[src_v.at[b]], hbel_v.at[b], gsem[b])
        pltpu.async_copy(er_hbm.at[dst_v.at[b]], er_v.at[b], gsem[b])

    def wait_gathers(b):
        pltpu.make_async_copy(hbel_hbm.at[src_v.at[b]], hbel_v.at[b],
                              gsem[b]).wait()
        pltpu.make_async_copy(er_hbm.at[dst_v.at[b]], er_v.at[b],
                              gsem[b]).wait()

    def wait_scatters(b):
        pltpu.make_async_copy(msg_v.at[b], acc_s.at[sdst_v.at[b]],
                              ssem[b]).wait()

    def compute(b):
        hb, mb = hbel_v.at[b], msg_v.at[b]
        erb = er_v.at[b]

        def edge(k, carry):
            el, _ = plsc.unpack(hb[k, pl.ds(D, 32)],
                                format=plsc.PackFormat.INTERLEAVED)
            e = el + erb[k]
            e = jnp.where(e > 0, e, 0.2 * e)
            w = jnp.exp(e)
            mb[k, pl.ds(D, HD)] = w
            for q in range(NH // 2):
                ha, hb2 = plsc.unpack(hb[k, pl.ds(q * 32, 32)],
                                      format=plsc.PackFormat.INTERLEAVED)
                mb[k, pl.ds((2 * q) * HD, HD)] = ha * w[2 * q]
                mb[k, pl.ds((2 * q + 1) * HD, HD)] = hb2 * w[2 * q + 1]
            return carry

        lax.fori_loop(0, CHUNK, edge, 0, unroll=2)

    issue_gathers(0, 0)
    issue_gathers(1, 1)

    def pair(p, carry):
        for b in range(2):
            c = 2 * p + b
            wait_gathers(b)

            @pl.when(p > 0)
            def _():
                wait_scatters(b)

            compute(b)
            for i in range(CHUNK // 16):
                sdst_v.at[b][pl.ds(i * 16, 16)] = dst_v.at[b][pl.ds(i * 16, 16)]
            pltpu.async_copy(msg_v.at[b], acc_s.at[sdst_v.at[b]], ssem[b],
                             add=True)

            @pl.when(c + 2 < nchunks)
            def _():
                issue_gathers(c + 2, b)
        return carry

    lax.fori_loop(0, nchunks // 2, pair, 0)
    for b in range(2):
        wait_scatters(b)
    plsc.subcore_barrier()

    @pl.when(cid == 0)
    def _():
        pltpu.sync_copy(acc_s.at[pl.ds(sid * ZR, ZR)],
                        acc0_out.at[pl.ds(sid * ZR, ZR)])

    @pl.when(cid == 1)
    def _():
        pltpu.sync_copy(acc_s.at[pl.ds(sid * ZR, ZR)],
                        acc1_out.at[pl.ds(sid * ZR, ZR)])


def _sc_edge(hbel, er_dst, src_idx, dst_idx):
    mesh = plsc.VectorSubcoreMesh(core_axis_name="c", subcore_axis_name="s")
    f = pl.kernel(
        _sc_edge_body,
        mesh=mesh,
        compiler_params=pltpu.CompilerParams(use_tc_tiling_on_sc=False,
                                             needs_layout_passes=False),
        out_type=[
            jax.ShapeDtypeStruct((N_ACC, AW), jnp.float32),
            jax.ShapeDtypeStruct((N_ACC, AW), jnp.float32),
        ],
        scratch_types=[
            pltpu.VMEM((2, CHUNK), jnp.int32),
            pltpu.VMEM((2, CHUNK), jnp.int32),
            pltpu.VMEM((2, CHUNK), jnp.int32),
            pltpu.VMEM((2, CHUNK, 16), jnp.float32),
            pltpu.VMEM((2, CHUNK, D + 32), jnp.bfloat16),
            pltpu.VMEM((2, CHUNK, AW), jnp.float32),
            pltpu.VMEM_SHARED((N_ACC, AW), jnp.float32),
            pltpu.SemaphoreType.DMA,
            pltpu.SemaphoreType.DMA,
            pltpu.SemaphoreType.DMA,
            pltpu.SemaphoreType.DMA,
        ],
    )
    return f(hbel, er_dst, src_idx, dst_idx)


def _ln(x, g, b):
    mu = jnp.mean(x, axis=-1, keepdims=True)
    var = jnp.mean((x - mu) ** 2, axis=-1, keepdims=True)
    return (x - mu) / jnp.sqrt(var + 1e-5) * g + b


def _fin_body(acc0_ref, acc1_ref, hd_ref, eld_ref, erd_ref, fd_ref,
              wvs_ref, bvs_ref, wos_ref, bos_ref,
              wq_ref, bq_ref, wk_ref, bk_ref, wv_ref, bv_ref, wo_ref, bo_ref,
              g1_ref, b1_ref, g2_ref, b2_ref, g3_ref, b3_ref,
              w1_ref, bf1_ref, w2_ref, bf2_ref,
              erep_ref, ered_ref,
              out_ref, gat_ref):
    erep = erep_ref[...]
    ered = ered_ref[...]

    eself = eld_ref[...] + erd_ref[...]
    eself = jnp.where(eself > 0, eself, 0.2 * eself)
    wself = jnp.exp(eself)
    wrep = jnp.dot(wself, erep, preferred_element_type=jnp.float32)
    den16 = acc0_ref[:, D:] + acc1_ref[:, D:]
    den = jnp.dot(den16, erep, preferred_element_type=jnp.float32) + wrep
    num = acc0_ref[:, :D] + acc1_ref[:, :D] + wrep * hd_ref[...]
    gat = num / den
    gat_ref[...] = gat

    t = fd_ref[...]
    sa = jnp.dot(jnp.dot(t, wvs_ref[...], preferred_element_type=jnp.float32),
                 wos_ref[...], preferred_element_type=jnp.float32)
    sa = sa + bvs_ref[...] @ wos_ref[...] + bos_ref[...]
    x = _ln(t + sa, g1_ref[...], b1_ref[...])

    q = jnp.dot(x, wq_ref[...], preferred_element_type=jnp.float32) + bq_ref[...]
    k1 = jnp.dot(t, wk_ref[...], preferred_element_type=jnp.float32) + bk_ref[...]
    k2 = jnp.dot(gat, wk_ref[...], preferred_element_type=jnp.float32) + bk_ref[...]
    v1 = jnp.dot(t, wv_ref[...], preferred_element_type=jnp.float32) + bv_ref[...]
    v2 = jnp.dot(gat, wv_ref[...], preferred_element_type=jnp.float32) + bv_ref[...]
    s1 = jnp.dot(q * k1, ered, preferred_element_type=jnp.float32)
    s2 = jnp.dot(q * k2, ered, preferred_element_type=jnp.float32)
    a1 = 1.0 / (1.0 + jnp.exp(s2 - s1))
    a1r = jnp.dot(a1, erep, preferred_element_type=jnp.float32)
    o = v2 + a1r * (v1 - v2)
    ca = jnp.dot(o, wo_ref[...], preferred_element_type=jnp.float32) + bo_ref[...]
    x = _ln(x + ca, g2_ref[...], b2_ref[...])

    hid = jnp.maximum(
        jnp.dot(x, w1_ref[...], preferred_element_type=jnp.float32)
        + bf1_ref[...], 0.0)
    ff = jnp.dot(hid, w2_ref[...], preferred_element_type=jnp.float32) + bf2_ref[...]
    out_ref[...] = _ln(x + ff, g3_ref[...], b3_ref[...])


def _finalize(acc0, acc1, h_dst, el_dst, er_dst, feat_dst, weights):
    nb = 25
    B = N_DST // nb

    def row(shape):
        return pl.BlockSpec(shape, lambda i: (i, 0))

    def full(shape):
        return pl.BlockSpec(shape, lambda i: (0, 0))

    in_specs = [
        row((B, AW)), row((B, AW)),
        row((B, D)), row((B, 16)), row((B, 16)), row((B, D)),
    ]
    weight_shapes = [w.shape for w in weights]
    in_specs += [full(s) for s in weight_shapes]
    return pl.pallas_call(
        _fin_body,
        grid=(nb,),
        in_specs=in_specs,
        out_specs=[row((B, D)), row((B, D))],
        out_shape=[
            jax.ShapeDtypeStruct((N_DST, D), jnp.float32),
            jax.ShapeDtypeStruct((N_DST, D), jnp.float32),
        ],
    )(acc0, acc1, h_dst, el_dst, er_dst, feat_dst, *weights)


def kernel(feat_src, feat_dst, params, edge_index):
    rows = jnp.arange(D)
    colmask = (rows[:, None] // HD) == jnp.arange(16)[None, :]
    almat = jnp.where(colmask, params["attn_l"].reshape(-1)[:, None], 0.0)
    armat = jnp.where(colmask, params["attn_r"].reshape(-1)[:, None], 0.0)
    erep = colmask.astype(jnp.float32).T
    ered = colmask.astype(jnp.float32) * 0.25

    wv = rows // 32
    rem = rows % 32
    half = rem // 16
    ii = rem % 16
    pcol = 32 * wv + 2 * ii + half
    permat = (pcol[:, None] == rows[None, :]).astype(jnp.float32)

    c32 = jnp.arange(32)
    al32 = jnp.where(
        ((rows[:, None] // HD) == (c32[None, :] // 2)) & (c32[None, :] % 2 == 0),
        params["attn_l"].reshape(-1)[:, None], 0.0)

    pad = E_PAD - E
    src_p = jnp.concatenate(
        [edge_index[0].astype(jnp.int32), jnp.zeros((pad,), jnp.int32)])
    dst_p = jnp.concatenate(
        [edge_index[1].astype(jnp.int32),
         jnp.full((pad,), N_DST, jnp.int32)])

    hbel = _gat_pre_src(feat_src, params["W_gat"], al32, permat)
    h_dst, el_dst, er_dst = _gat_pre_dst(feat_dst, params["W_gat"], almat,
                                         armat)

    acc0, acc1 = _sc_edge(hbel, er_dst, src_p, dst_p)

    sa_p, ca_p = params["sa"], params["ca"]

    def r1(v):
        return v.reshape(1, -1)

    weights = [
        sa_p["Wv"], r1(sa_p["bv"]), sa_p["Wo"], r1(sa_p["bo"]),
        ca_p["Wq"], r1(ca_p["bq"]), ca_p["Wk"], r1(ca_p["bk"]),
        ca_p["Wv"], r1(ca_p["bv"]), ca_p["Wo"], r1(ca_p["bo"]),
        r1(params["ln1_g"]), r1(params["ln1_b"]),
        r1(params["ln2_g"]), r1(params["ln2_b"]),
        r1(params["ln3_g"]), r1(params["ln3_b"]),
        params["W1"], r1(params["b1"]), params["W2"], r1(params["b2"]),
        erep, ered,
    ]
    out, gat = _finalize(acc0, acc1, h_dst, el_dst, er_dst, feat_dst, weights)
    return out, gat


# speedup vs baseline: 73.0146x; 1.1666x over previous
import functools

import jax
import jax.numpy as jnp
from jax import lax
from jax.experimental import pallas as pl
from jax.experimental.pallas import tpu as pltpu
from jax.experimental.pallas import tpu_sc as plsc

D = 128
NH = 8
HD = 16
N_SRC = 10000
N_DST = 10000
E = 320000
FF = 2048

NW = 32
CHUNK = 64
SUP = 512
EPW = 10240
E_PAD = NW * EPW
N_ACC = 10112
ZR = N_ACC // 16
AW = D + HD


def _asrc_body(x_ref, w_ref, al32_ref, pm_ref, hbel_ref):
    h = jnp.dot(x_ref[...], w_ref[...], preferred_element_type=jnp.float32)
    hp = jnp.dot(h, pm_ref[...], preferred_element_type=jnp.float32)
    elp = jnp.dot(h, al32_ref[...], preferred_element_type=jnp.float32)
    hbel_ref[...] = jnp.concatenate(
        [hp.astype(jnp.bfloat16), elp.astype(jnp.bfloat16)], axis=-1)


def _gat_pre_src(x, w_gat, al32, permat):
    nb = 5
    rb = N_SRC // nb
    return pl.pallas_call(
        _asrc_body,
        grid=(nb,),
        in_specs=[
            pl.BlockSpec((rb, D), lambda i: (i, 0)),
            pl.BlockSpec((D, D), lambda i: (0, 0)),
            pl.BlockSpec((D, 32), lambda i: (0, 0)),
            pl.BlockSpec((D, D), lambda i: (0, 0)),
        ],
        out_specs=pl.BlockSpec((rb, D + 32), lambda i: (i, 0)),
        out_shape=jax.ShapeDtypeStruct((N_SRC, D + 32), jnp.bfloat16),
    )(x, w_gat, al32, permat)


def _adst_body(x_ref, w_ref, al_ref, ar_ref, h_ref, el_ref, er_ref):
    h = jnp.dot(x_ref[...], w_ref[...], preferred_element_type=jnp.float32)
    h_ref[...] = h
    el_ref[...] = jnp.dot(h, al_ref[...], preferred_element_type=jnp.float32)
    er_ref[...] = jnp.dot(h, ar_ref[...], preferred_element_type=jnp.float32)


def _gat_pre_dst(x, w_gat, almat, armat):
    nb = 5
    rb = N_DST // nb
    return pl.pallas_call(
        _adst_body,
        grid=(nb,),
        in_specs=[
            pl.BlockSpec((rb, D), lambda i: (i, 0)),
            pl.BlockSpec((D, D), lambda i: (0, 0)),
            pl.BlockSpec((D, 16), lambda i: (0, 0)),
            pl.BlockSpec((D, 16), lambda i: (0, 0)),
        ],
        out_specs=[
            pl.BlockSpec((rb, D), lambda i: (i, 0)),
            pl.BlockSpec((rb, 16), lambda i: (i, 0)),
            pl.BlockSpec((rb, 16), lambda i: (i, 0)),
        ],
        out_shape=[
            jax.ShapeDtypeStruct((N_DST, D), jnp.float32),
            jax.ShapeDtypeStruct((N_DST, 16), jnp.float32),
            jax.ShapeDtypeStruct((N_DST, 16), jnp.float32),
        ],
    )(x, w_gat, almat, armat)


def _sc_edge_body(hbel_hbm, er_hbm, src_hbm, dst_hbm,
                  acc0_out, acc1_out,
                  sidx, didx, sdst_v, er_v, hbel_v, msg_v,
                  acc_s,
                  gsem0, gsem1, ssem0, ssem1, isem0, isem1):
    cid = lax.axis_index("c")
    sid = lax.axis_index("s")
    wid = sid * 2 + cid
    gsem = (gsem0, gsem1)
    ssem = (ssem0, ssem1)
    isem = (isem0, isem1)

    zv = jnp.zeros((16,), jnp.float32)
    zb = msg_v.at[0]

    def zrow(r, carry):
        for j in range(AW // 16):
            zb[r, pl.ds(j * 16, 16)] = zv
        return carry

    lax.fori_loop(0, CHUNK, zrow, 0, unroll=4)
    base = sid * ZR
    for j in range(ZR // CHUNK):
        pltpu.sync_copy(zb, acc_s.at[pl.ds(base + j * CHUNK, CHUNK)])
    rem_rows = ZR % CHUNK
    if rem_rows:
        pltpu.sync_copy(zb.at[pl.ds(0, rem_rows)],
                        acc_s.at[pl.ds(base + (ZR // CHUNK) * CHUNK,
                                       rem_rows)])
    plsc.subcore_barrier()

    nchunks = EPW // CHUNK
    cps = SUP // CHUNK
    nsup = EPW // SUP

    def issue_idx(s, B):
        base = wid * EPW + s * SUP
        pltpu.async_copy(src_hbm.at[pl.ds(base, SUP)], sidx.at[B], isem[B])
        pltpu.async_copy(dst_hbm.at[pl.ds(base, SUP)], didx.at[B], isem[B])

    def wait_idx(s, B):
        base = wid * EPW + s * SUP
        pltpu.make_async_copy(src_hbm.at[pl.ds(base, SUP)], sidx.at[B],
                              isem[B]).wait()
        pltpu.make_async_copy(dst_hbm.at[pl.ds(base, SUP)], didx.at[B],
                              isem[B]).wait()

    def issue_gathers(j, B, b):
        src_sl = sidx.at[B, pl.ds(j * CHUNK, CHUNK)]
        dst_sl = didx.at[B, pl.ds(j * CHUNK, CHUNK)]
        pltpu.async_copy(hbel_hbm.at[src_sl], hbel_v.at[b], gsem[b])
        pltpu.async_copy(er_hbm.at[dst_sl], er_v.at[b], gsem[b])

    def wait_gathers(j, B, b):
        src_sl = sidx.at[B, pl.ds(j * CHUNK, CHUNK)]
        dst_sl = didx.at[B, pl.ds(j * CHUNK, CHUNK)]
        pltpu.make_async_copy(hbel_hbm.at[src_sl], hbel_v.at[b],
                              gsem[b]).wait()
        pltpu.make_async_copy(er_hbm.at[dst_sl], er_v.at[b],
                              gsem[b]).wait()

    def wait_scatters(b):
        pltpu.make_async_copy(msg_v.at[b], acc_s.at[sdst_v.at[b]],
                              ssem[b]).wait()

    def compute(b):
        hb, mb = hbel_v.at[b], msg_v.at[b]
        erb = er_v.at[b]

        def edge(k, carry):
            el, _ = plsc.unpack(hb[k, pl.ds(D, 32)],
                                format=plsc.PackFormat.INTERLEAVED)
            e = el + erb[k]
            e = jnp.where(e > 0, e, 0.2 * e)
            w = jnp.exp(e)
            mb[k, pl.ds(D, HD)] = w
            for q in range(NH // 2):
                ha, hb2 = plsc.unpack(hb[k, pl.ds(q * 32, 32)],
                                      format=plsc.PackFormat.INTERLEAVED)
                mb[k, pl.ds((2 * q) * HD, HD)] = ha * w[2 * q]
                mb[k, pl.ds((2 * q + 1) * HD, HD)] = hb2 * w[2 * q + 1]
            return carry

        lax.fori_loop(0, CHUNK, edge, 0, unroll=2)

    issue_idx(0, 0)
    wait_idx(0, 0)
    issue_gathers(0, 0, 0)
    issue_gathers(1, 0, 1)

    def super_pair(sp, carry):
        for S in range(2):
            s = 2 * sp + S

            @pl.when(s + 1 < nsup)
            def _():
                issue_idx(s + 1, 1 - S)

            for j in range(cps):
                b = j % 2
                c = s * cps + j
                wait_gathers(j, S, b)

                @pl.when(c >= 2)
                def _():
                    wait_scatters(b)

                for i in range(CHUNK // 16):
                    sdst_v.at[b][pl.ds(i * 16, 16)] = (
                        didx.at[S][pl.ds(j * CHUNK + i * 16, 16)])
                compute(b)
                pltpu.async_copy(msg_v.at[b], acc_s.at[sdst_v.at[b]],
                                 ssem[b], add=True)

                if j == cps - 2:
                    @pl.when(s + 1 < nsup)
                    def _():
                        wait_idx(s + 1, 1 - S)

                jn = j + 2
                Bn, jn = (S, jn) if jn < cps else (1 - S, jn - cps)

                @pl.when(c + 2 < nchunks)
                def _():
                    issue_gathers(jn, Bn, b)
        return carry

    lax.fori_loop(0, nsup // 2, super_pair, 0)
    for b in range(2):
        wait_scatters(b)
    plsc.subcore_barrier()

    @pl.when(cid == 0)
    def _():
        pltpu.sync_copy(acc_s.at[pl.ds(sid * ZR, ZR)],
                        acc0_out.at[pl.ds(sid * ZR, ZR)])

    @pl.when(cid == 1)
    def _():
        pltpu.sync_copy(acc_s.at[pl.ds(sid * ZR, ZR)],
                        acc1_out.at[pl.ds(sid * ZR, ZR)])


def _sc_edge(hbel, er_dst, src_idx, dst_idx):
    mesh = plsc.VectorSubcoreMesh(core_axis_name="c", subcore_axis_name="s")
    f = pl.kernel(
        _sc_edge_body,
        mesh=mesh,
        compiler_params=pltpu.CompilerParams(use_tc_tiling_on_sc=False,
                                             needs_layout_passes=False),
        out_type=[
            jax.ShapeDtypeStruct((N_ACC, AW), jnp.float32),
            jax.ShapeDtypeStruct((N_ACC, AW), jnp.float32),
        ],
        scratch_types=[
            pltpu.VMEM((2, SUP), jnp.int32),
            pltpu.VMEM((2, SUP), jnp.int32),
            pltpu.VMEM((2, CHUNK), jnp.int32),
            pltpu.VMEM((2, CHUNK, 16), jnp.float32),
            pltpu.VMEM((2, CHUNK, D + 32), jnp.bfloat16),
            pltpu.VMEM((2, CHUNK, AW), jnp.float32),
            pltpu.VMEM_SHARED((N_ACC, AW), jnp.float32),
            pltpu.SemaphoreType.DMA,
            pltpu.SemaphoreType.DMA,
            pltpu.SemaphoreType.DMA,
            pltpu.SemaphoreType.DMA,
            pltpu.SemaphoreType.DMA,
            pltpu.SemaphoreType.DMA,
        ],
    )
    return f(hbel, er_dst, src_idx, dst_idx)


def _ln(x, g, b):
    mu = jnp.mean(x, axis=-1, keepdims=True)
    var = jnp.mean((x - mu) ** 2, axis=-1, keepdims=True)
    return (x - mu) / jnp.sqrt(var + 1e-5) * g + b


def _fin_body(acc0_ref, acc1_ref, hd_ref, eld_ref, erd_ref, fd_ref,
              wvs_ref, bvs_ref, wos_ref, bos_ref,
              wq_ref, bq_ref, wk_ref, bk_ref, wv_ref, bv_ref, wo_ref, bo_ref,
              g1_ref, b1_ref, g2_ref, b2_ref, g3_ref, b3_ref,
              w1_ref, bf1_ref, w2_ref, bf2_ref,
              erep_ref, ered_ref,
              out_ref, gat_ref):
    erep = erep_ref[...]
    ered = ered_ref[...]

    eself = eld_ref[...] + erd_ref[...]
    eself = jnp.where(eself > 0, eself, 0.2 * eself)
    wself = jnp.exp(eself)
    wrep = jnp.dot(wself, erep, preferred_element_type=jnp.float32)
    den16 = acc0_ref[:, D:] + acc1_ref[:, D:]
    den = jnp.dot(den16, erep, preferred_element_type=jnp.float32) + wrep
    num = acc0_ref[:, :D] + acc1_ref[:, :D] + wrep * hd_ref[...]
    gat = num / den
    gat_ref[...] = gat

    t = fd_ref[...]
    sa = jnp.dot(jnp.dot(t, wvs_ref[...], preferred_element_type=jnp.float32),
                 wos_ref[...], preferred_element_type=jnp.float32)
    sa = sa + bvs_ref[...] @ wos_ref[...] + bos_ref[...]
    x = _ln(t + sa, g1_ref[...], b1_ref[...])

    q = jnp.dot(x, wq_ref[...], preferred_element_type=jnp.float32) + bq_ref[...]
    k1 = jnp.dot(t, wk_ref[...], preferred_element_type=jnp.float32) + bk_ref[...]
    k2 = jnp.dot(gat, wk_ref[...], preferred_element_type=jnp.float32) + bk_ref[...]
    v1 = jnp.dot(t, wv_ref[...], preferred_element_type=jnp.float32) + bv_ref[...]
    v2 = jnp.dot(gat, wv_ref[...], preferred_element_type=jnp.float32) + bv_ref[...]
    s1 = jnp.dot(q * k1, ered, preferred_element_type=jnp.float32)
    s2 = jnp.dot(q * k2, ered, preferred_element_type=jnp.float32)
    a1 = 1.0 / (1.0 + jnp.exp(s2 - s1))
    a1r = jnp.dot(a1, erep, preferred_element_type=jnp.float32)
    o = v2 + a1r * (v1 - v2)
    ca = jnp.dot(o, wo_ref[...], preferred_element_type=jnp.float32) + bo_ref[...]
    x = _ln(x + ca, g2_ref[...], b2_ref[...])

    hid = jnp.maximum(
        jnp.dot(x, w1_ref[...], preferred_element_type=jnp.float32)
        + bf1_ref[...], 0.0)
    ff = jnp.dot(hid, w2_ref[...], preferred_element_type=jnp.float32) + bf2_ref[...]
    out_ref[...] = _ln(x + ff, g3_ref[...], b3_ref[...])


def _finalize(acc0, acc1, h_dst, el_dst, er_dst, feat_dst, weights):
    nb = 25
    B = N_DST // nb

    def row(shape):
        return pl.BlockSpec(shape, lambda i: (i, 0))

    def full(shape):
        return pl.BlockSpec(shape, lambda i: (0, 0))

    in_specs = [
        row((B, AW)), row((B, AW)),
        row((B, D)), row((B, 16)), row((B, 16)), row((B, D)),
    ]
    weight_shapes = [w.shape for w in weights]
    in_specs += [full(s) for s in weight_shapes]
    return pl.pallas_call(
        _fin_body,
        grid=(nb,),
        in_specs=in_specs,
        out_specs=[row((B, D)), row((B, D))],
        out_shape=[
            jax.ShapeDtypeStruct((N_DST, D), jnp.float32),
            jax.ShapeDtypeStruct((N_DST, D), jnp.float32),
        ],
    )(acc0, acc1, h_dst, el_dst, er_dst, feat_dst, *weights)


def kernel(feat_src, feat_dst, params, edge_index):
    rows = jnp.arange(D)
    colmask = (rows[:, None] // HD) == jnp.arange(16)[None, :]
    almat = jnp.where(colmask, params["attn_l"].reshape(-1)[:, None], 0.0)
    armat = jnp.where(colmask, params["attn_r"].reshape(-1)[:, None], 0.0)
    erep = colmask.astype(jnp.float32).T
    ered = colmask.astype(jnp.float32) * 0.25

    wv = rows // 32
    rem = rows % 32
    half = rem // 16
    ii = rem % 16
    pcol = 32 * wv + 2 * ii + half
    permat = (pcol[:, None] == rows[None, :]).astype(jnp.float32)

    c32 = jnp.arange(32)
    al32 = jnp.where(
        ((rows[:, None] // HD) == (c32[None, :] // 2)) & (c32[None, :] % 2 == 0),
        params["attn_l"].reshape(-1)[:, None], 0.0)

    pad = E_PAD - E
    src_p = jnp.concatenate(
        [edge_index[0].astype(jnp.int32), jnp.zeros((pad,), jnp.int32)])
    dst_p = jnp.concatenate(
        [edge_index[1].astype(jnp.int32),
         jnp.full((pad,), N_DST, jnp.int32)])

    hbel = _gat_pre_src(feat_src, params["W_gat"], al32, permat)
    h_dst, el_dst, er_dst = _gat_pre_dst(feat_dst, params["W_gat"], almat,
                                         armat)

    acc0, acc1 = _sc_edge(hbel, er_dst, src_p, dst_p)

    sa_p, ca_p = params["sa"], params["ca"]

    def r1(v):
        return v.reshape(1, -1)

    weights = [
        sa_p["Wv"], r1(sa_p["bv"]), sa_p["Wo"], r1(sa_p["bo"]),
        ca_p["Wq"], r1(ca_p["bq"]), ca_p["Wk"], r1(ca_p["bk"]),
        ca_p["Wv"], r1(ca_p["bv"]), ca_p["Wo"], r1(ca_p["bo"]),
        r1(params["ln1_g"]), r1(params["ln1_b"]),
        r1(params["ln2_g"]), r1(params["ln2_b"]),
        r1(params["ln3_g"]), r1(params["ln3_b"]),
        params["W1"], r1(params["b1"]), params["W2"], r1(params["b2"]),
        erep, ered,
    ]
    out, gat = _finalize(acc0, acc1, h_dst, el_dst, er_dst, feat_dst, weights)
    return out, gat
